# Initial kernel scaffold; baseline (speedup 1.0000x reference)
import functools

import jax
import jax.numpy as jnp
from jax import lax
from jax.experimental import pallas as pl
from jax.experimental.pallas import tpu as pltpu
from jax.experimental.pallas import tpu_sc as plsc

NC = 2
NS = 16
L = 16
CH = 128


def _sc_mesh():
  return plsc.VectorSubcoreMesh(core_axis_name="c", subcore_axis_name="s",
                                num_cores=NC, num_subcores=NS)


def _deg_call(N, E):
  nw = NC * NS
  epw = E // nw
  nfull, rem = divmod(epw, CH)
  rpt = N // NS

  @functools.partial(
      pl.kernel,
      out_type=jax.ShapeDtypeStruct((NC, N, L), jnp.float32),
      mesh=_sc_mesh(),
      scratch_types=[
          pltpu.VMEM((CH,), jnp.int32),
          pltpu.VMEM((max(rem, 8),), jnp.int32),
          pltpu.VMEM((CH, L), jnp.float32),
          pltpu.VMEM_SHARED((N, L), jnp.float32),
      ],
  )
  def k(col_hbm, ones_hbm, zeros_hbm, degp_hbm, colbuf, colbuf_r, ones_v, acc):
    cid = lax.axis_index("c")
    sid = lax.axis_index("s")
    wid = cid * NS + sid
    rbase = sid * rpt
    pltpu.sync_copy(ones_hbm, ones_v)
    pltpu.sync_copy(zeros_hbm, acc.at[pl.ds(rbase, rpt)])
    plsc.subcore_barrier()
    ebase = wid * epw

    def body(i, carry):
      off = ebase + i * CH
      pltpu.sync_copy(col_hbm.at[pl.ds(off, CH)], colbuf)
      pltpu.sync_copy(ones_v, acc.at[colbuf], add=True)
      return carry

    lax.fori_loop(0, nfull, body, 0)
    if rem:
      off = ebase + nfull * CH
      pltpu.sync_copy(col_hbm.at[pl.ds(off, rem)], colbuf_r)
      pltpu.sync_copy(ones_v.at[pl.ds(0, rem)], acc.at[colbuf_r], add=True)
    plsc.subcore_barrier()
    pltpu.sync_copy(acc.at[pl.ds(rbase, rpt)],
                    degp_hbm.at[cid, pl.ds(rbase, rpt)])

  return k


def _edge_call(N, E, F):
  nw = NC * NS
  epw = E // nw
  nfull, rem = divmod(epw, CH)
  rpt = N // NS

  @functools.partial(
      pl.kernel,
      out_type=jax.ShapeDtypeStruct((NC, N, F), jnp.float32),
      mesh=_sc_mesh(),
      scratch_types=[
          pltpu.VMEM((CH,), jnp.int32),
          pltpu.VMEM((CH,), jnp.int32),
          pltpu.VMEM((max(rem, 8),), jnp.int32),
          pltpu.VMEM((max(rem, 8),), jnp.int32),
          pltpu.VMEM((CH, F), jnp.float32),
          pltpu.VMEM_SHARED((N, F), jnp.float32),
          pltpu.SemaphoreType.DMA,
      ],
  )
  def k(xs_hbm, row_hbm, col_hbm, zeros_hbm, accp_hbm,
        rowbuf, colbuf, rowbuf_r, colbuf_r, rows_v, acc, sem):
    cid = lax.axis_index("c")
    sid = lax.axis_index("s")
    wid = cid * NS + sid
    rbase = sid * rpt
    pltpu.sync_copy(zeros_hbm, acc.at[pl.ds(rbase, rpt)])
    plsc.subcore_barrier()
    ebase = wid * epw

    def body(i, carry):
      off = ebase + i * CH
      pltpu.sync_copy(row_hbm.at[pl.ds(off, CH)], rowbuf)
      pltpu.sync_copy(col_hbm.at[pl.ds(off, CH)], colbuf)
      pltpu.async_copy(xs_hbm.at[rowbuf], rows_v, sem).wait()
      pltpu.sync_copy(rows_v, acc.at[colbuf], add=True)
      return carry

    lax.fori_loop(0, nfull, body, 0)
    if rem:
      off = ebase + nfull * CH
      pltpu.sync_copy(row_hbm.at[pl.ds(off, rem)], rowbuf_r)
      pltpu.sync_copy(col_hbm.at[pl.ds(off, rem)], colbuf_r)
      pltpu.async_copy(xs_hbm.at[rowbuf_r], rows_v.at[pl.ds(0, rem)], sem).wait()
      pltpu.sync_copy(rows_v.at[pl.ds(0, rem)], acc.at[colbuf_r], add=True)
    plsc.subcore_barrier()
    pltpu.sync_copy(acc.at[pl.ds(rbase, rpt)],
                    accp_hbm.at[cid, pl.ds(rbase, rpt)])

  return k


def _b1_call(N, F, H, BN):
  def body(x_ref, w_ref, degp_ref, xs_ref, dinv_ref):
    degp = degp_ref[...]
    deg = 1.0 + degp[0, :, 0] + degp[1, :, 0]
    di = lax.rsqrt(deg)
    xw = jnp.dot(x_ref[...], w_ref[...], preferred_element_type=jnp.float32)
    xs_ref[...] = xw * di[:, None]
    dinv_ref[...] = di[:, None]

  return pl.pallas_call(
      body,
      grid=(N // BN,),
      in_specs=[
          pl.BlockSpec((BN, F), lambda i: (i, 0)),
          pl.BlockSpec((F, H), lambda i: (0, 0)),
          pl.BlockSpec((NC, BN, L), lambda i: (0, i, 0)),
      ],
      out_specs=[
          pl.BlockSpec((BN, H), lambda i: (i, 0)),
          pl.BlockSpec((BN, 1), lambda i: (i, 0)),
      ],
      out_shape=[
          jax.ShapeDtypeStruct((N, H), jnp.float32),
          jax.ShapeDtypeStruct((N, 1), jnp.float32),
      ],
  )


def _b2_call(N, H, O, BN):
  def body(accp_ref, xs1_ref, dinv_ref, b1_ref, w2_ref, xs2_ref):
    s = accp_ref[0] + accp_ref[1] + xs1_ref[...]
    h = jnp.maximum(s * dinv_ref[...] + b1_ref[...], 0.0)
    xs2_ref[...] = jnp.dot(h, w2_ref[...],
                           preferred_element_type=jnp.float32) * dinv_ref[...]

  return pl.pallas_call(
      body,
      grid=(N // BN,),
      in_specs=[
          pl.BlockSpec((NC, BN, H), lambda i: (0, i, 0)),
          pl.BlockSpec((BN, H), lambda i: (i, 0)),
          pl.BlockSpec((BN, 1), lambda i: (i, 0)),
          pl.BlockSpec((1, H), lambda i: (0, 0)),
          pl.BlockSpec((H, O), lambda i: (0, 0)),
      ],
      out_specs=pl.BlockSpec((BN, O), lambda i: (i, 0)),
      out_shape=jax.ShapeDtypeStruct((N, O), jnp.float32),
  )


def _b3_call(N, O, BN):
  def body(accp_ref, xs2_ref, dinv_ref, b2_ref, out_ref):
    s = accp_ref[0] + accp_ref[1] + xs2_ref[...]
    out_ref[...] = s * dinv_ref[...] + b2_ref[...]

  return pl.pallas_call(
      body,
      grid=(N // BN,),
      in_specs=[
          pl.BlockSpec((NC, BN, O), lambda i: (0, i, 0)),
          pl.BlockSpec((BN, O), lambda i: (i, 0)),
          pl.BlockSpec((BN, 1), lambda i: (i, 0)),
          pl.BlockSpec((1, O), lambda i: (0, 0)),
      ],
      out_specs=pl.BlockSpec((BN, O), lambda i: (i, 0)),
      out_shape=jax.ShapeDtypeStruct((N, O), jnp.float32),
  )


def kernel(x, edge_index, edge_attr, W1, b1, W2, b2):
  N, F = x.shape
  H = W1.shape[1]
  O = W2.shape[1]
  E = edge_index.shape[1]
  del edge_attr
  assert E % (NC * NS) == 0 and N % NS == 0
  BN = 1000
  assert N % BN == 0

  row = edge_index[0]
  col = edge_index[1]
  rpt = N // NS
  ones16 = jnp.ones((CH, L), jnp.float32)
  zeros16 = jnp.zeros((rpt, L), jnp.float32)
  zerosF = jnp.zeros((rpt, F), jnp.float32)

  degp = _deg_call(N, E)(col, ones16, zeros16)
  xs1, dinv = _b1_call(N, F, H, BN)(x, W1, degp)
  accp1 = _edge_call(N, E, H)(xs1, row, col, zerosF)
  xs2 = _b2_call(N, H, O, BN)(accp1, xs1, dinv, b1.reshape(1, H), W2)
  accp2 = _edge_call(N, E, O)(xs2, row, col, zerosF)
  out = _b3_call(N, O, BN)(accp2, xs2, dinv, b2.reshape(1, O))
  return out


# speedup vs baseline: 17.5342x; 17.5342x over previous
import functools

import jax
import jax.numpy as jnp
from jax import lax
from jax.experimental import pallas as pl
from jax.experimental.pallas import tpu as pltpu
from jax.experimental.pallas import tpu_sc as plsc

def _round8(v):
  return (v + 7) // 8 * 8


NC = 2
NS = 16
L = 16
CH = 128


def _sc_mesh():
  return plsc.VectorSubcoreMesh(core_axis_name="c", subcore_axis_name="s",
                                num_cores=NC, num_subcores=NS)


def _deg_call(N, E):
  nw = NC * NS
  epw = E // nw
  rpt = _round8(-(-N // NS))
  NP = rpt * NS

  @functools.partial(
      pl.kernel,
      out_type=jax.ShapeDtypeStruct((nw, NP), jnp.float32),
      mesh=_sc_mesh(),
      scratch_types=[
          pltpu.VMEM((epw,), jnp.int32),
          pltpu.VMEM((NP,), jnp.float32),
      ],
      compiler_params=pltpu.CompilerParams(needs_layout_passes=False),
  )
  def k(col_hbm, degp_hbm, colstage, hist):
    cid = lax.axis_index("c")
    sid = lax.axis_index("s")
    wid = cid * NS + sid

    def zbody(i, c):
      hist[pl.ds(i * L, L)] = jnp.zeros(L, jnp.float32)
      return c

    lax.fori_loop(0, NP // L, zbody, 0)
    pltpu.sync_copy(col_hbm.at[pl.ds(wid * epw, epw)], colstage)
    ones = jnp.ones(L, jnp.float32)

    def body(i, c):
      idx = colstage[pl.ds(i * L, L)]
      plsc.addupdate_scatter(hist, [idx], ones)
      return c

    lax.fori_loop(0, epw // L, body, 0)
    pltpu.sync_copy(hist, degp_hbm.at[pl.ds(wid, 1)].at[0])

  return k


def _edge_call(N, E, F):
  nw = NC * NS
  epw = E // nw
  nfull, rem = divmod(epw, CH)
  rpt = _round8(-(-N // NS))
  NP = rpt * NS

  @functools.partial(
      pl.kernel,
      out_type=jax.ShapeDtypeStruct((NC, NP, F), jnp.float32),
      mesh=_sc_mesh(),
      scratch_types=[
          pltpu.VMEM((CH,), jnp.int32),
          pltpu.VMEM((CH,), jnp.int32),
          pltpu.VMEM((max(rem, 8),), jnp.int32),
          pltpu.VMEM((max(rem, 8),), jnp.int32),
          pltpu.VMEM((CH, F), jnp.float32),
          pltpu.VMEM_SHARED((NP, F), jnp.float32),
          pltpu.SemaphoreType.DMA,
      ],
  )
  def k(xs_hbm, row_hbm, col_hbm, zeros_hbm, accp_hbm,
        rowbuf, colbuf, rowbuf_r, colbuf_r, rows_v, acc, sem):
    cid = lax.axis_index("c")
    sid = lax.axis_index("s")
    wid = cid * NS + sid
    rbase = sid * rpt
    pltpu.sync_copy(zeros_hbm, acc.at[pl.ds(rbase, rpt)])
    plsc.subcore_barrier()
    ebase = wid * epw

    def body(i, carry):
      off = ebase + i * CH
      pltpu.sync_copy(row_hbm.at[pl.ds(off, CH)], rowbuf)
      pltpu.sync_copy(col_hbm.at[pl.ds(off, CH)], colbuf)
      pltpu.async_copy(xs_hbm.at[rowbuf], rows_v, sem).wait()
      pltpu.sync_copy(rows_v, acc.at[colbuf], add=True)
      return carry

    lax.fori_loop(0, nfull, body, 0)
    if rem:
      off = ebase + nfull * CH
      pltpu.sync_copy(row_hbm.at[pl.ds(off, rem)], rowbuf_r)
      pltpu.sync_copy(col_hbm.at[pl.ds(off, rem)], colbuf_r)
      pltpu.async_copy(xs_hbm.at[rowbuf_r], rows_v.at[pl.ds(0, rem)], sem).wait()
      pltpu.sync_copy(rows_v.at[pl.ds(0, rem)], acc.at[colbuf_r], add=True)
    plsc.subcore_barrier()
    pltpu.sync_copy(acc.at[pl.ds(rbase, rpt)],
                    accp_hbm.at[cid, pl.ds(rbase, rpt)])

  return k


def _b1_call(N, F, H, BN):
  def body(x_ref, w_ref, degp_ref, xs_ref, dinv_ref):
    deg = 1.0 + jnp.sum(degp_ref[...], axis=0)
    di = lax.rsqrt(deg)
    xw = jnp.dot(x_ref[...], w_ref[...], preferred_element_type=jnp.float32)
    xs_ref[...] = xw * di[:, None]
    dinv_ref[...] = di[:, None]

  return pl.pallas_call(
      body,
      grid=(pl.cdiv(N, BN),),
      in_specs=[
          pl.BlockSpec((BN, F), lambda i: (i, 0)),
          pl.BlockSpec((F, H), lambda i: (0, 0)),
          pl.BlockSpec((NC * NS, BN), lambda i: (0, i)),
      ],
      out_specs=[
          pl.BlockSpec((BN, H), lambda i: (i, 0)),
          pl.BlockSpec((BN, 1), lambda i: (i, 0)),
      ],
      out_shape=[
          jax.ShapeDtypeStruct((N, H), jnp.float32),
          jax.ShapeDtypeStruct((N, 1), jnp.float32),
      ],
  )


def _b2_call(N, H, O, BN):
  def body(accp_ref, xs1_ref, dinv_ref, b1_ref, w2_ref, xs2_ref):
    s = accp_ref[0] + accp_ref[1] + xs1_ref[...]
    h = jnp.maximum(s * dinv_ref[...] + b1_ref[...], 0.0)
    xs2_ref[...] = jnp.dot(h, w2_ref[...],
                           preferred_element_type=jnp.float32) * dinv_ref[...]

  return pl.pallas_call(
      body,
      grid=(pl.cdiv(N, BN),),
      in_specs=[
          pl.BlockSpec((NC, BN, H), lambda i: (0, i, 0)),
          pl.BlockSpec((BN, H), lambda i: (i, 0)),
          pl.BlockSpec((BN, 1), lambda i: (i, 0)),
          pl.BlockSpec((1, H), lambda i: (0, 0)),
          pl.BlockSpec((H, O), lambda i: (0, 0)),
      ],
      out_specs=pl.BlockSpec((BN, O), lambda i: (i, 0)),
      out_shape=jax.ShapeDtypeStruct((N, O), jnp.float32),
  )


def _b3_call(N, O, BN):
  def body(accp_ref, xs2_ref, dinv_ref, b2_ref, out_ref):
    s = accp_ref[0] + accp_ref[1] + xs2_ref[...]
    out_ref[...] = s * dinv_ref[...] + b2_ref[...]

  return pl.pallas_call(
      body,
      grid=(pl.cdiv(N, BN),),
      in_specs=[
          pl.BlockSpec((NC, BN, O), lambda i: (0, i, 0)),
          pl.BlockSpec((BN, O), lambda i: (i, 0)),
          pl.BlockSpec((BN, 1), lambda i: (i, 0)),
          pl.BlockSpec((1, O), lambda i: (0, 0)),
      ],
      out_specs=pl.BlockSpec((BN, O), lambda i: (i, 0)),
      out_shape=jax.ShapeDtypeStruct((N, O), jnp.float32),
  )


def kernel(x, edge_index, edge_attr, W1, b1, W2, b2):
  N, F = x.shape
  H = W1.shape[1]
  O = W2.shape[1]
  E = edge_index.shape[1]
  del edge_attr
  assert E % (NC * NS) == 0 and N % NS == 0
  BN = 1024

  row = edge_index[0]
  col = edge_index[1]
  rpt = _round8(-(-N // NS))
  zerosF = jnp.zeros((rpt, F), jnp.float32)

  degp = _deg_call(N, E)(col)
  xs1, dinv = _b1_call(N, F, H, BN)(x, W1, degp)
  accp1 = _edge_call(N, E, H)(xs1, row, col, zerosF)
  xs2 = _b2_call(N, H, O, BN)(accp1, xs1, dinv, b1.reshape(1, H), W2)
  accp2 = _edge_call(N, E, O)(xs2, row, col, zerosF)
  out = _b3_call(N, O, BN)(accp2, xs2, dinv, b2.reshape(1, O))
  return out
